# Initial kernel scaffold; baseline (speedup 1.0000x reference)
import functools

import jax
import jax.numpy as jnp
from jax import lax
from jax.experimental import pallas as pl
from jax.experimental.pallas import tpu as pltpu
from jax.experimental.pallas import tpu_sc as plsc

L = 16


def _make_lookup(N, V, num_cores, num_subcores):
    NW = num_cores * num_subcores
    per_w = N // NW
    mesh = plsc.VectorSubcoreMesh(core_axis_name="c", subcore_axis_name="s")

    @functools.partial(
        pl.kernel,
        mesh=mesh,
        out_type=jax.ShapeDtypeStruct((N,), jnp.int32),
        scratch_types=[
            pltpu.VMEM((V,), jnp.int32),
            pltpu.VMEM((per_w,), jnp.int32),
            pltpu.VMEM((per_w,), jnp.int32),
        ],
    )
    def k(tok_hbm, tab_hbm, out_hbm, tab_v, tok_v, out_v):
        wid = lax.axis_index("s") * num_cores + lax.axis_index("c")
        base = wid * per_w
        pltpu.sync_copy(tab_hbm, tab_v)
        pltpu.sync_copy(tok_hbm.at[pl.ds(base, per_w)], tok_v)

        def body(i, carry):
            t = tok_v[pl.ds(i * L, L)]
            idx = lax.shift_right_logical(t, 1)
            out_v[pl.ds(i * L, L)] = plsc.load_gather(tab_v, [idx])
            return carry

        lax.fori_loop(0, per_w // L, body, 0)
        pltpu.sync_copy(out_v, out_hbm.at[pl.ds(base, per_w)])

    return k


def kernel(tokens, table):
    B, S = tokens.shape
    N = B * S
    V = table.shape[0]
    info = plsc.get_sparse_core_info()
    k = _make_lookup(N, V, info.num_cores, info.num_subcores)
    out = k(tokens.reshape(N), table)
    return out.reshape(B, S)


# speedup vs baseline: 174.9318x; 174.9318x over previous
import functools

import jax
import jax.numpy as jnp
from jax import lax
from jax.experimental import pallas as pl
from jax.experimental.pallas import tpu as pltpu
from jax.experimental.pallas import tpu_sc as plsc

L = 16


def _make_lookup(N, V, num_cores, num_subcores):
    NW = num_cores * num_subcores
    per_w = N // NW
    mesh = plsc.VectorSubcoreMesh(core_axis_name="c", subcore_axis_name="s")

    @functools.partial(
        pl.kernel,
        mesh=mesh,
        out_type=jax.ShapeDtypeStruct((N,), jnp.int32),
        scratch_types=[
            pltpu.VMEM((V,), jnp.int32),
            pltpu.VMEM((per_w,), jnp.int32),
            pltpu.VMEM((per_w,), jnp.int32),
        ],
        compiler_params=pltpu.CompilerParams(needs_layout_passes=False),
    )
    def k(tok_hbm, tab_hbm, out_hbm, tab_v, tok_v, out_v):
        wid = lax.axis_index("s") * num_cores + lax.axis_index("c")
        base = wid * per_w
        pltpu.sync_copy(tab_hbm, tab_v)
        pltpu.sync_copy(tok_hbm.at[pl.ds(base, per_w)], tok_v)

        def body(i, carry):
            t = tok_v[pl.ds(i * L, L)]
            idx = lax.shift_right_logical(t, 1)
            out_v[pl.ds(i * L, L)] = plsc.load_gather(tab_v, [idx])
            return carry

        lax.fori_loop(0, per_w // L, body, 0)
        pltpu.sync_copy(out_v, out_hbm.at[pl.ds(base, per_w)])

    return k


def kernel(tokens, table):
    B, S = tokens.shape
    N = B * S
    V = table.shape[0]
    info = plsc.get_sparse_core_info()
    k = _make_lookup(N, V, info.num_cores, info.num_subcores)
    out = k(tokens.reshape(N), table)
    return out.reshape(B, S)


# speedup vs baseline: 195.8879x; 1.1198x over previous
import functools

import jax
import jax.numpy as jnp
from jax import lax
from jax.experimental import pallas as pl
from jax.experimental.pallas import tpu as pltpu
from jax.experimental.pallas import tpu_sc as plsc

L = 16


def _make_lookup(N, V, num_cores, num_subcores):
    NW = num_cores * num_subcores
    per_w = N // NW
    mesh = plsc.VectorSubcoreMesh(core_axis_name="c", subcore_axis_name="s")

    @functools.partial(
        pl.kernel,
        mesh=mesh,
        out_type=jax.ShapeDtypeStruct((N,), jnp.int32),
        scratch_types=[
            pltpu.VMEM((V,), jnp.int32),
            pltpu.VMEM((per_w,), jnp.int32),
            pltpu.VMEM((per_w,), jnp.int32),
            pltpu.SemaphoreType.DMA,
            pltpu.SemaphoreType.DMA,
        ],
        compiler_params=pltpu.CompilerParams(needs_layout_passes=False),
    )
    def k(tok_hbm, tab_hbm, out_hbm, tab_v, tok_v, out_v, sem_tab, sem_tok):
        wid = lax.axis_index("s") * num_cores + lax.axis_index("c")
        base = wid * per_w
        tab_cp = pltpu.async_copy(tab_hbm, tab_v, sem_tab)
        tok_cp = pltpu.async_copy(tok_hbm.at[pl.ds(base, per_w)], tok_v, sem_tok)
        tab_cp.wait()
        tok_cp.wait()

        @plsc.parallel_loop(0, per_w, L, unroll=8)
        def body(i):
            t = tok_v[pl.ds(i, L)]
            idx = lax.shift_right_logical(t, 1)
            out_v[pl.ds(i, L)] = plsc.load_gather(tab_v, [idx])

        pltpu.sync_copy(out_v, out_hbm.at[pl.ds(base, per_w)])

    return k


def kernel(tokens, table):
    B, S = tokens.shape
    N = B * S
    V = table.shape[0]
    info = plsc.get_sparse_core_info()
    k = _make_lookup(N, V, info.num_cores, info.num_subcores)
    out = k(tokens.reshape(N), table)
    return out.reshape(B, S)


# speedup vs baseline: 197.7492x; 1.0095x over previous
import functools

import jax
import jax.numpy as jnp
from jax import lax
from jax.experimental import pallas as pl
from jax.experimental.pallas import tpu as pltpu
from jax.experimental.pallas import tpu_sc as plsc

L = 16


def _make_lookup(N, V, num_cores, num_subcores):
    NW = num_cores * num_subcores
    per_w = N // NW
    mesh = plsc.VectorSubcoreMesh(core_axis_name="c", subcore_axis_name="s")

    @functools.partial(
        pl.kernel,
        mesh=mesh,
        out_type=jax.ShapeDtypeStruct((N,), jnp.int32),
        scratch_types=[
            pltpu.VMEM((V,), jnp.int32),
            pltpu.VMEM((per_w,), jnp.int32),
            pltpu.VMEM((per_w,), jnp.int32),
            pltpu.SemaphoreType.DMA,
            pltpu.SemaphoreType.DMA,
            pltpu.SemaphoreType.DMA,
        ],
        compiler_params=pltpu.CompilerParams(needs_layout_passes=False),
    )
    def k(tok_hbm, tab_hbm, out_hbm, tab_v, tok_v, out_v, sem_tab, sem_tok, sem_out):
        wid = lax.axis_index("s") * num_cores + lax.axis_index("c")
        base = wid * per_w
        tok_cp = pltpu.async_copy(tok_hbm.at[pl.ds(base, per_w)], tok_v, sem_tok)
        tab_cp = pltpu.async_copy(tab_hbm, tab_v, sem_tab)
        tab_cp.wait()
        tok_cp.wait()

        nchunk = 4
        chunk = per_w // nchunk
        out_cps = []
        for c in range(nchunk):
            off = c * chunk

            @plsc.parallel_loop(off, off + chunk, L, unroll=8)
            def body(i):
                t = tok_v[pl.ds(i, L)]
                idx = lax.shift_right_logical(t, 1)
                out_v[pl.ds(i, L)] = plsc.load_gather(tab_v, [idx])

            out_cps.append(
                pltpu.async_copy(
                    out_v.at[pl.ds(off, chunk)],
                    out_hbm.at[pl.ds(base + off, chunk)],
                    sem_out,
                )
            )
        for cp in out_cps:
            cp.wait()

    return k


def kernel(tokens, table):
    B, S = tokens.shape
    N = B * S
    V = table.shape[0]
    info = plsc.get_sparse_core_info()
    k = _make_lookup(N, V, info.num_cores, info.num_subcores)
    out = k(tokens.reshape(N), table)
    return out.reshape(B, S)


# speedup vs baseline: 244.8136x; 1.2380x over previous
import functools

import jax
import jax.numpy as jnp
from jax import lax
from jax.experimental import pallas as pl
from jax.experimental.pallas import tpu as pltpu
from jax.experimental.pallas import tpu_sc as plsc

L = 16


def _make_lookup(R, C, V, num_cores, num_subcores):
    NW = num_cores * num_subcores
    rows_w = R // NW
    ncol = (C + L - 1) // L
    col_off = [j * L for j in range(C // L)]
    if C % L:
        col_off.append(C - L)
    mesh = plsc.VectorSubcoreMesh(core_axis_name="c", subcore_axis_name="s")

    @functools.partial(
        pl.kernel,
        mesh=mesh,
        out_type=jax.ShapeDtypeStruct((R, C), jnp.int32),
        scratch_types=[
            pltpu.VMEM((V,), jnp.int32),
            pltpu.VMEM((rows_w, C), jnp.int32),
            pltpu.VMEM((rows_w, C), jnp.int32),
            pltpu.SemaphoreType.DMA,
            pltpu.SemaphoreType.DMA,
            pltpu.SemaphoreType.DMA,
        ],
        compiler_params=pltpu.CompilerParams(needs_layout_passes=False),
    )
    def k(tok_hbm, tab_hbm, out_hbm, tab_v, tok_v, out_v, sem_tab, sem_tok, sem_out):
        wid = lax.axis_index("s") * num_cores + lax.axis_index("c")
        base = wid * rows_w
        tok_cp = pltpu.async_copy(tok_hbm.at[pl.ds(base, rows_w)], tok_v, sem_tok)
        tab_cp = pltpu.async_copy(tab_hbm, tab_v, sem_tab)
        tab_cp.wait()
        tok_cp.wait()

        nchunk = 4
        rchunk = rows_w // nchunk
        out_cps = []
        for ci in range(nchunk):
            r0 = ci * rchunk

            @plsc.parallel_loop(r0, r0 + rchunk, 1, unroll=1)
            def body(r):
                for c in col_off:
                    t = tok_v[r, pl.ds(c, L)]
                    idx = lax.shift_right_logical(t, 1)
                    out_v[r, pl.ds(c, L)] = plsc.load_gather(tab_v, [idx])

            out_cps.append(
                pltpu.async_copy(
                    out_v.at[pl.ds(r0, rchunk)],
                    out_hbm.at[pl.ds(base + r0, rchunk)],
                    sem_out,
                )
            )
        for cp in out_cps:
            cp.wait()

    return k


def kernel(tokens, table):
    R, C = tokens.shape
    V = table.shape[0]
    info = plsc.get_sparse_core_info()
    k = _make_lookup(R, C, V, info.num_cores, info.num_subcores)
    return k(tokens, table)


# speedup vs baseline: 245.5788x; 1.0031x over previous
import functools

import jax
import jax.numpy as jnp
from jax import lax
from jax.experimental import pallas as pl
from jax.experimental.pallas import tpu as pltpu
from jax.experimental.pallas import tpu_sc as plsc

L = 16


def _make_lookup(R, C, V, num_cores, num_subcores):
    NW = num_cores * num_subcores
    rows_w = R // NW
    ncol = (C + L - 1) // L
    col_off = [j * L for j in range(C // L)]
    if C % L:
        col_off.append(C - L)
    mesh = plsc.VectorSubcoreMesh(core_axis_name="c", subcore_axis_name="s")

    @functools.partial(
        pl.kernel,
        mesh=mesh,
        out_type=jax.ShapeDtypeStruct((R, C), jnp.int32),
        scratch_types=[
            pltpu.VMEM((V,), jnp.int32),
            pltpu.VMEM((rows_w, C), jnp.int32),
            pltpu.VMEM((rows_w, C), jnp.int32),
            pltpu.SemaphoreType.DMA,
            pltpu.SemaphoreType.DMA,
            pltpu.SemaphoreType.DMA,
        ],
        compiler_params=pltpu.CompilerParams(
            needs_layout_passes=False, use_tc_tiling_on_sc=True
        ),
    )
    def k(tok_hbm, tab_hbm, out_hbm, tab_v, tok_v, out_v, sem_tab, sem_tok, sem_out):
        wid = lax.axis_index("s") * num_cores + lax.axis_index("c")
        base = wid * rows_w
        tok_cp = pltpu.async_copy(tok_hbm.at[pl.ds(base, rows_w)], tok_v, sem_tok)
        tab_cp = pltpu.async_copy(tab_hbm, tab_v, sem_tab)
        tab_cp.wait()
        tok_cp.wait()

        nchunk = 4
        rchunk = rows_w // nchunk
        out_cps = []
        for ci in range(nchunk):
            r0 = ci * rchunk

            @plsc.parallel_loop(r0, r0 + rchunk, 1, unroll=1)
            def body(r):
                for c in col_off:
                    t = tok_v[r, pl.ds(c, L)]
                    idx = lax.shift_right_logical(t, 1)
                    out_v[r, pl.ds(c, L)] = plsc.load_gather(tab_v, [idx])

            out_cps.append(
                pltpu.async_copy(
                    out_v.at[pl.ds(r0, rchunk)],
                    out_hbm.at[pl.ds(base + r0, rchunk)],
                    sem_out,
                )
            )
        for cp in out_cps:
            cp.wait()

    return k


def kernel(tokens, table):
    R, C = tokens.shape
    V = table.shape[0]
    info = plsc.get_sparse_core_info()
    k = _make_lookup(R, C, V, info.num_cores, info.num_subcores)
    return k(tokens, table)


# speedup vs baseline: 302.6895x; 1.2326x over previous
import functools

import jax
import jax.numpy as jnp
from jax import lax
from jax.experimental import pallas as pl
from jax.experimental.pallas import tpu as pltpu
from jax.experimental.pallas import tpu_sc as plsc

L = 16


def _make_lookup(R, C, V, num_cores, num_subcores):
    NW = num_cores * num_subcores
    cols_w = C // NW
    vregs_row = cols_w // L
    mesh = plsc.VectorSubcoreMesh(core_axis_name="c", subcore_axis_name="s")

    @functools.partial(
        pl.kernel,
        mesh=mesh,
        out_type=jax.ShapeDtypeStruct((R, C), jnp.int32),
        scratch_types=[
            pltpu.VMEM((V,), jnp.int32),
            pltpu.VMEM((R, cols_w), jnp.int32),
            pltpu.VMEM((R, cols_w), jnp.int32),
            pltpu.SemaphoreType.DMA,
            pltpu.SemaphoreType.DMA,
            pltpu.SemaphoreType.DMA,
        ],
        compiler_params=pltpu.CompilerParams(needs_layout_passes=False),
    )
    def k(tok_hbm, tab_hbm, out_hbm, tab_v, tok_v, out_v, sem_tab, sem_tok, sem_out):
        wid = lax.axis_index("s") * num_cores + lax.axis_index("c")
        c0 = wid * cols_w
        tok_cp = pltpu.async_copy(tok_hbm.at[:, pl.ds(c0, cols_w)], tok_v, sem_tok)
        tab_cp = pltpu.async_copy(tab_hbm, tab_v, sem_tab)
        tab_cp.wait()
        tok_cp.wait()

        nchunk = 5
        rchunk = R // nchunk
        out_cps = []
        for ci in range(nchunk):
            r0 = ci * rchunk

            @plsc.parallel_loop(r0, r0 + rchunk, 1, unroll=2)
            def body(r):
                for j in range(vregs_row):
                    t = tok_v[r, pl.ds(j * L, L)]
                    idx = lax.shift_right_logical(t, 1)
                    out_v[r, pl.ds(j * L, L)] = plsc.load_gather(tab_v, [idx])

            out_cps.append(
                pltpu.async_copy(
                    out_v.at[pl.ds(r0, rchunk)],
                    out_hbm.at[pl.ds(r0, rchunk), pl.ds(c0, cols_w)],
                    sem_out,
                )
            )
        for cp in out_cps:
            cp.wait()

    return k


def kernel(tokens, table):
    R, C = tokens.shape
    V = table.shape[0]
    info = plsc.get_sparse_core_info()
    k = _make_lookup(C, R, V, info.num_cores, info.num_subcores)
    out_t = k(tokens.T, table)
    return out_t.T
